# baseline (speedup 1.0000x reference)
import jax
import jax.numpy as jnp
from jax import lax
from jax.experimental import pallas as pl
from jax.experimental.pallas import tpu as pltpu
from jax.experimental.pallas import tpu_sc as plsc

NUM_CORES = 2
NUM_SUBCORES = 16
LANES = 16
NW = NUM_CORES * NUM_SUBCORES

BATCH = 16384
EMBED_DIM = 64
B_PER_W = BATCH // NW
CHUNK = 32
N_CHUNKS = B_PER_W // CHUNK
GROUPS = CHUNK // LANES


def _body(cidx_hbm, aidx_hbm, cpair_hbm, apair_hbm, out_hbm,
          cidx_v, aidx_v, ctid_v, atid_v, cbuf_v, abuf_v, out_v, sem):
    wid = lax.axis_index("s") * NUM_CORES + lax.axis_index("c")
    base = wid * N_CHUNKS

    pltpu.sync_copy(cidx_hbm.at[pl.ds(base, N_CHUNKS)], cidx_v)
    pltpu.sync_copy(aidx_hbm.at[pl.ds(base, N_CHUNKS)], aidx_v)

    def tid_body(k, carry):
        for g in range(GROUPS):
            sl = pl.ds(g * LANES, LANES)
            ctid_v[k, sl] = lax.shift_right_logical(cidx_v[k, sl], 1)
            atid_v[k, sl] = lax.shift_right_logical(aidx_v[k, sl], 1)
        return carry

    lax.fori_loop(0, N_CHUNKS, tid_body, 0)

    lane = lax.iota(jnp.int32, LANES)
    perms = [(lane ^ m).reshape(LANES, 1) for m in (8, 4, 2, 1)]
    dnums = lax.GatherDimensionNumbers(
        offset_dims=(), collapsed_slice_dims=(0,), start_index_map=(0,))

    def shuffle(x, p):
        return lax.gather(x, p, dnums, slice_sizes=(1,),
                          mode=lax.GatherScatterMode.PROMISE_IN_BOUNDS)

    def chunk_body(k, carry):
        ca = pltpu.async_copy(cpair_hbm.at[ctid_v.at[k]], cbuf_v, sem)
        aa = pltpu.async_copy(apair_hbm.at[atid_v.at[k]], abuf_v, sem)
        ca.wait()
        aa.wait()

        for g in range(GROUPS):
            sl = pl.ds(g * LANES, LANES)
            cov = (cidx_v[k, sl] & 1) * EMBED_DIM
            aov = (aidx_v[k, sl] & 1) * EMBED_DIM
            out_vec = jnp.zeros((LANES,), jnp.float32)
            for l in range(LANES):
                j = g * LANES + l
                co = cov[l]
                ao = aov[l]
                acc = (cbuf_v[j, pl.ds(co, LANES)]
                       * abuf_v[j, pl.ds(ao, LANES)])
                for d in range(1, EMBED_DIM // LANES):
                    acc = acc + (cbuf_v[j, pl.ds(co + d * LANES, LANES)]
                                 * abuf_v[j, pl.ds(ao + d * LANES, LANES)])
                for p in perms:
                    acc = acc + shuffle(acc, p)
                out_vec = jnp.where(lane == l, acc, out_vec)
            out_v[pl.ds(k * CHUNK + g * LANES, LANES)] = out_vec
        return carry

    lax.fori_loop(0, N_CHUNKS, chunk_body, 0)

    pltpu.sync_copy(out_v, out_hbm.at[pl.ds(wid * B_PER_W, B_PER_W)])


@jax.jit
def kernel(customer, article, customer_table, article_table):
    mesh = plsc.VectorSubcoreMesh(core_axis_name="c", subcore_axis_name="s")
    run = pl.kernel(
        _body,
        out_type=jax.ShapeDtypeStruct((BATCH,), jnp.float32),
        mesh=mesh,
        scratch_types=[
            pltpu.VMEM((N_CHUNKS, CHUNK), jnp.int32),
            pltpu.VMEM((N_CHUNKS, CHUNK), jnp.int32),
            pltpu.VMEM((N_CHUNKS, CHUNK), jnp.int32),
            pltpu.VMEM((N_CHUNKS, CHUNK), jnp.int32),
            pltpu.VMEM((CHUNK, 2 * EMBED_DIM), jnp.float32),
            pltpu.VMEM((CHUNK, 2 * EMBED_DIM), jnp.float32),
            pltpu.VMEM((B_PER_W,), jnp.float32),
            pltpu.SemaphoreType.DMA,
        ],
    )
    cidx = customer.reshape(NW * N_CHUNKS, CHUNK)
    aidx = article.reshape(NW * N_CHUNKS, CHUNK)
    cpair = jnp.concatenate(
        [customer_table[0::2], customer_table[1::2]], axis=1)
    apair = jnp.concatenate(
        [article_table[0::2], article_table[1::2]], axis=1)
    return run(cidx, aidx, cpair, apair)


# speedup vs baseline: 23.2177x; 23.2177x over previous
import jax
import jax.numpy as jnp
from jax import lax
from jax.experimental import pallas as pl
from jax.experimental.pallas import tpu as pltpu
from jax.experimental.pallas import tpu_sc as plsc

NUM_CORES = 2
NUM_SUBCORES = 16
LANES = 16
NW = NUM_CORES * NUM_SUBCORES

BATCH = 16384
EMBED_DIM = 64
B_PER_W = BATCH // NW
CHUNK = 32
N_CHUNKS = B_PER_W // CHUNK
GROUPS = CHUNK // LANES
NBUF = 2


def _body(cidx_hbm, aidx_hbm, ctable_hbm, atable_hbm, out_hbm,
          cidx_v, aidx_v, cbuf_v, abuf_v, out_v, *sems):
    wid = lax.axis_index("s") * NUM_CORES + lax.axis_index("c")
    base = wid * N_CHUNKS

    pltpu.sync_copy(cidx_hbm.at[pl.ds(base, N_CHUNKS)], cidx_v)
    pltpu.sync_copy(aidx_hbm.at[pl.ds(base, N_CHUNKS)], aidx_v)

    lane = lax.iota(jnp.int32, LANES)
    perms = [(lane ^ m).reshape(LANES, 1) for m in (8, 4, 2, 1)]
    dnums = lax.GatherDimensionNumbers(
        offset_dims=(), collapsed_slice_dims=(0,), start_index_map=(0,))

    def shuffle(x, p):
        return lax.gather(x, p, dnums, slice_sizes=(1,),
                          mode=lax.GatherScatterMode.PROMISE_IN_BOUNDS)

    def fire(k, b):
        copies = []
        for g in range(GROUPS):
            civ = cidx_v[k, pl.ds(g * LANES, LANES)]
            aiv = aidx_v[k, pl.ds(g * LANES, LANES)]
            for l in range(LANES):
                j = g * LANES + l
                copies.append(pltpu.async_copy(
                    ctable_hbm.at[pl.ds(civ[l], 1)],
                    cbuf_v.at[b, pl.ds(j, 1)], sems[b]))
                copies.append(pltpu.async_copy(
                    atable_hbm.at[pl.ds(aiv[l], 1)],
                    abuf_v.at[b, pl.ds(j, 1)], sems[b]))
        return copies

    def drain(copies):
        for c in copies:
            c.wait()

    def compute(k, b):
        for g in range(GROUPS):
            out_vec = jnp.zeros((LANES,), jnp.float32)
            for l in range(LANES):
                j = g * LANES + l
                acc = (cbuf_v[b, j, pl.ds(0, LANES)]
                       * abuf_v[b, j, pl.ds(0, LANES)])
                for d in range(1, EMBED_DIM // LANES):
                    acc = acc + (cbuf_v[b, j, pl.ds(d * LANES, LANES)]
                                 * abuf_v[b, j, pl.ds(d * LANES, LANES)])
                for p in perms:
                    acc = acc + shuffle(acc, p)
                out_vec = jnp.where(lane == l, acc, out_vec)
            out_v[pl.ds(k * CHUNK + g * LANES, LANES)] = out_vec

    drain(fire(0, 0))

    def loop_body(k2, carry):
        for b in range(NBUF):
            k = k2 * NBUF + b
            nxt = fire(k + 1, (b + 1) % NBUF)
            compute(k, b)
            drain(nxt)
        return carry

    lax.fori_loop(0, (N_CHUNKS - 1) // NBUF, loop_body, 0)

    k = N_CHUNKS - 2
    nxt = fire(k + 1, 1)
    compute(k, 0)
    drain(nxt)
    compute(N_CHUNKS - 1, 1)

    pltpu.sync_copy(out_v, out_hbm.at[pl.ds(wid * B_PER_W, B_PER_W)])


@jax.jit
def kernel(customer, article, customer_table, article_table):
    mesh = plsc.VectorSubcoreMesh(core_axis_name="c", subcore_axis_name="s")
    run = pl.kernel(
        _body,
        out_type=jax.ShapeDtypeStruct((BATCH,), jnp.float32),
        mesh=mesh,
        scratch_types=[
            pltpu.VMEM((N_CHUNKS, CHUNK), jnp.int32),
            pltpu.VMEM((N_CHUNKS, CHUNK), jnp.int32),
            pltpu.VMEM((NBUF, CHUNK, EMBED_DIM), jnp.float32),
            pltpu.VMEM((NBUF, CHUNK, EMBED_DIM), jnp.float32),
            pltpu.VMEM((B_PER_W,), jnp.float32),
        ] + [pltpu.SemaphoreType.DMA] * NBUF,
    )
    cidx = customer.reshape(NW * N_CHUNKS, CHUNK)
    aidx = article.reshape(NW * N_CHUNKS, CHUNK)
    return run(cidx, aidx, customer_table, article_table)


# speedup vs baseline: 23.5444x; 1.0141x over previous
import jax
import jax.numpy as jnp
from jax import lax
from jax.experimental import pallas as pl
from jax.experimental.pallas import tpu as pltpu
from jax.experimental.pallas import tpu_sc as plsc

NUM_CORES = 2
NUM_SUBCORES = 16
LANES = 16
NW = NUM_CORES * NUM_SUBCORES

BATCH = 16384
EMBED_DIM = 64
B_PER_W = BATCH // NW
CHUNK = 32
N_CHUNKS = B_PER_W // CHUNK
GROUPS = CHUNK // LANES


def _body(cidx_hbm, aidx_hbm, ctable_hbm, atable_hbm, out_hbm,
          cidx_v, aidx_v, cbuf_v, abuf_v, out_v, *sems):
    wid = lax.axis_index("s") * NUM_CORES + lax.axis_index("c")
    base = wid * N_CHUNKS

    pltpu.sync_copy(cidx_hbm.at[pl.ds(base, N_CHUNKS)], cidx_v)
    pltpu.sync_copy(aidx_hbm.at[pl.ds(base, N_CHUNKS)], aidx_v)

    lane = lax.iota(jnp.int32, LANES)
    perms = [(lane ^ m).reshape(LANES, 1) for m in (8, 4, 2, 1)]
    dnums = lax.GatherDimensionNumbers(
        offset_dims=(), collapsed_slice_dims=(0,), start_index_map=(0,))

    def shuffle(x, p):
        return lax.gather(x, p, dnums, slice_sizes=(1,),
                          mode=lax.GatherScatterMode.PROMISE_IN_BOUNDS)

    def chunk_body(k, carry):
        copies = []
        for g in range(GROUPS):
            civ = cidx_v[k, pl.ds(g * LANES, LANES)]
            aiv = aidx_v[k, pl.ds(g * LANES, LANES)]
            for l in range(LANES):
                j = g * LANES + l
                copies.append(pltpu.async_copy(
                    ctable_hbm.at[pl.ds(civ[l], 1)],
                    cbuf_v.at[pl.ds(j, 1)], sems[(2 * j) % len(sems)]))
                copies.append(pltpu.async_copy(
                    atable_hbm.at[pl.ds(aiv[l], 1)],
                    abuf_v.at[pl.ds(j, 1)], sems[(2 * j + 1) % len(sems)]))
        for c in copies:
            c.wait()

        for g in range(GROUPS):
            out_vec = jnp.zeros((LANES,), jnp.float32)
            for l in range(LANES):
                j = g * LANES + l
                acc = (cbuf_v[j, pl.ds(0, LANES)]
                       * abuf_v[j, pl.ds(0, LANES)])
                for d in range(1, EMBED_DIM // LANES):
                    acc = acc + (cbuf_v[j, pl.ds(d * LANES, LANES)]
                                 * abuf_v[j, pl.ds(d * LANES, LANES)])
                for p in perms:
                    acc = acc + shuffle(acc, p)
                out_vec = jnp.where(lane == l, acc, out_vec)
            out_v[pl.ds(k * CHUNK + g * LANES, LANES)] = out_vec
        return carry

    lax.fori_loop(0, N_CHUNKS, chunk_body, 0)

    pltpu.sync_copy(out_v, out_hbm.at[pl.ds(wid * B_PER_W, B_PER_W)])


@jax.jit
def kernel(customer, article, customer_table, article_table):
    mesh = plsc.VectorSubcoreMesh(core_axis_name="c", subcore_axis_name="s")
    run = pl.kernel(
        _body,
        out_type=jax.ShapeDtypeStruct((BATCH,), jnp.float32),
        mesh=mesh,
        scratch_types=[
            pltpu.VMEM((N_CHUNKS, CHUNK), jnp.int32),
            pltpu.VMEM((N_CHUNKS, CHUNK), jnp.int32),
            pltpu.VMEM((CHUNK, EMBED_DIM), jnp.float32),
            pltpu.VMEM((CHUNK, EMBED_DIM), jnp.float32),
            pltpu.VMEM((B_PER_W,), jnp.float32),
        ] + [pltpu.SemaphoreType.DMA] * 8,
    )
    cidx = customer.reshape(NW * N_CHUNKS, CHUNK)
    aidx = article.reshape(NW * N_CHUNKS, CHUNK)
    return run(cidx, aidx, customer_table, article_table)


# speedup vs baseline: 23.5580x; 1.0006x over previous
import jax
import jax.numpy as jnp
from jax import lax
from jax.experimental import pallas as pl
from jax.experimental.pallas import tpu as pltpu
from jax.experimental.pallas import tpu_sc as plsc

NUM_CORES = 2
NUM_SUBCORES = 16
LANES = 16
NW = NUM_CORES * NUM_SUBCORES

BATCH = 16384
EMBED_DIM = 64
B_PER_W = BATCH // NW
CHUNK = 128
N_CHUNKS = B_PER_W // CHUNK
GROUPS = CHUNK // LANES


def _body(cidx_hbm, aidx_hbm, ctable_hbm, atable_hbm, out_hbm,
          cidx_v, aidx_v, cbuf_v, abuf_v, out_v, *sems):
    wid = lax.axis_index("s") * NUM_CORES + lax.axis_index("c")
    base = wid * N_CHUNKS

    pltpu.sync_copy(cidx_hbm.at[pl.ds(base, N_CHUNKS)], cidx_v)
    pltpu.sync_copy(aidx_hbm.at[pl.ds(base, N_CHUNKS)], aidx_v)

    lane = lax.iota(jnp.int32, LANES)
    perms = [(lane ^ m).reshape(LANES, 1) for m in (8, 4, 2, 1)]
    dnums = lax.GatherDimensionNumbers(
        offset_dims=(), collapsed_slice_dims=(0,), start_index_map=(0,))

    def shuffle(x, p):
        return lax.gather(x, p, dnums, slice_sizes=(1,),
                          mode=lax.GatherScatterMode.PROMISE_IN_BOUNDS)

    def chunk_body(k, carry):
        copies = []
        for g in range(GROUPS):
            civ = cidx_v[k, pl.ds(g * LANES, LANES)]
            aiv = aidx_v[k, pl.ds(g * LANES, LANES)]
            for l in range(LANES):
                j = g * LANES + l
                copies.append(pltpu.async_copy(
                    ctable_hbm.at[pl.ds(civ[l], 1)],
                    cbuf_v.at[pl.ds(j, 1)], sems[(2 * j) % len(sems)]))
                copies.append(pltpu.async_copy(
                    atable_hbm.at[pl.ds(aiv[l], 1)],
                    abuf_v.at[pl.ds(j, 1)], sems[(2 * j + 1) % len(sems)]))
        for c in copies:
            c.wait()

        for g in range(GROUPS):
            out_vec = jnp.zeros((LANES,), jnp.float32)
            for l in range(LANES):
                j = g * LANES + l
                acc = (cbuf_v[j, pl.ds(0, LANES)]
                       * abuf_v[j, pl.ds(0, LANES)])
                for d in range(1, EMBED_DIM // LANES):
                    acc = acc + (cbuf_v[j, pl.ds(d * LANES, LANES)]
                                 * abuf_v[j, pl.ds(d * LANES, LANES)])
                for p in perms:
                    acc = acc + shuffle(acc, p)
                out_vec = jnp.where(lane == l, acc, out_vec)
            out_v[pl.ds(k * CHUNK + g * LANES, LANES)] = out_vec
        return carry

    lax.fori_loop(0, N_CHUNKS, chunk_body, 0)

    pltpu.sync_copy(out_v, out_hbm.at[pl.ds(wid * B_PER_W, B_PER_W)])


@jax.jit
def kernel(customer, article, customer_table, article_table):
    mesh = plsc.VectorSubcoreMesh(core_axis_name="c", subcore_axis_name="s")
    run = pl.kernel(
        _body,
        out_type=jax.ShapeDtypeStruct((BATCH,), jnp.float32),
        mesh=mesh,
        scratch_types=[
            pltpu.VMEM((N_CHUNKS, CHUNK), jnp.int32),
            pltpu.VMEM((N_CHUNKS, CHUNK), jnp.int32),
            pltpu.VMEM((CHUNK, EMBED_DIM), jnp.float32),
            pltpu.VMEM((CHUNK, EMBED_DIM), jnp.float32),
            pltpu.VMEM((B_PER_W,), jnp.float32),
        ] + [pltpu.SemaphoreType.DMA] * 8,
    )
    cidx = customer.reshape(NW * N_CHUNKS, CHUNK)
    aidx = article.reshape(NW * N_CHUNKS, CHUNK)
    return run(cidx, aidx, customer_table, article_table)
